# baseline (speedup 1.0000x reference)
import functools

import jax
import jax.numpy as jnp
from jax import lax
from jax.experimental import pallas as pl
from jax.experimental.pallas import tpu as pltpu
from jax.experimental.pallas import tpu_sc as plsc

N = 10000
D = 128
H = 64
OUT = 64
E = 320000

NC = 2
NS = 16
NW = NC * NS

NPAD = 10240
HALF = NPAD // NC
DUMP = 256
ACCR = HALF + DUMP
EPAD = 327680
EROWS = EPAD // 128
ROWS_PER_TILE = EROWS // NS
ZROWS = NPAD // NS
GSROWS = N // NS

_mesh = plsc.VectorSubcoreMesh(core_axis_name="c", subcore_axis_name="s")


@functools.partial(
    pl.kernel,
    out_type=jax.ShapeDtypeStruct((NC, NPAD, 16), jnp.float32),
    mesh=_mesh,
    scratch_types=[
        pltpu.VMEM((8, 128), jnp.int32),
        pltpu.VMEM((128, 16), jnp.float32),
        pltpu.VMEM((64, 16), jnp.float32),
        pltpu.VMEM_SHARED((NPAD, 16), jnp.float32),
    ],
    compiler_params=pltpu.CompilerParams(use_tc_tiling_on_sc=False),
)
def _deg_kernel(dst_hbm, out_hbm, dst_v, ones_v, zero_v, acc):
    cid = lax.axis_index("c")
    sid = lax.axis_index("s")
    wid = cid * NS + sid

    def fill_ones(i, carry):
        ones_v[i, :] = jnp.ones((16,), jnp.float32)
        return carry

    lax.fori_loop(0, 128, fill_ones, 0)

    def fill_zero(i, carry):
        zero_v[i, :] = jnp.zeros((16,), jnp.float32)
        return carry

    lax.fori_loop(0, 64, fill_zero, 0)

    def zero_acc(i, carry):
        pltpu.sync_copy(zero_v, acc.at[pl.ds(sid * ZROWS + i * 64, 64)])
        return carry

    lax.fori_loop(0, ZROWS // 64, zero_acc, 0)
    plsc.subcore_barrier()

    def chunk(c, carry):
        base = wid * (EROWS // NW) + c * 8
        pltpu.sync_copy(dst_hbm.at[pl.ds(base, 8)], dst_v)
        for j in range(8):
            pltpu.sync_copy(ones_v, acc.at[dst_v.at[j]], add=True)
        return carry

    lax.fori_loop(0, EROWS // NW // 8, chunk, 0)
    plsc.subcore_barrier()
    pltpu.sync_copy(acc.at[pl.ds(sid * ZROWS, ZROWS)],
                    out_hbm.at[cid, pl.ds(sid * ZROWS, ZROWS)])


_CR = 4
_NBUF = 2
_AZR = ACCR // NS


@functools.partial(
    pl.kernel,
    out_type=jax.ShapeDtypeStruct((NPAD, H), jnp.float32),
    mesh=_mesh,
    scratch_types=[
        pltpu.VMEM((_NBUF, _CR, 128), jnp.int32),
        pltpu.VMEM((_NBUF, _CR, 128), jnp.int32),
        pltpu.VMEM((_NBUF, _CR * 128, H), jnp.float32),
        pltpu.VMEM((16, H), jnp.float32),
        pltpu.VMEM_SHARED((ACCR, H), jnp.float32),
        pltpu.VMEM_SHARED((N, H), jnp.float32),
        pltpu.SemaphoreType.DMA,
        pltpu.SemaphoreType.DMA,
        pltpu.SemaphoreType.DMA,
    ],
    compiler_params=pltpu.CompilerParams(use_tc_tiling_on_sc=False),
)
def _msg_kernel(g_hbm, src_hbm, dst_hbm, out_hbm,
                src_v, dst_v, rows_v, zero_v, acc, g_sh, sem_g, sem_s, sem_t):
    cid = lax.axis_index("c")
    sid = lax.axis_index("s")
    base_node = cid * HALF

    stage = pltpu.async_copy(
        g_hbm.at[pl.ds(sid * GSROWS, GSROWS)],
        g_sh.at[pl.ds(sid * GSROWS, GSROWS)], sem_t)

    def fill_zero(i, carry):
        for j in range(H // 16):
            zero_v[i, pl.ds(j * 16, 16)] = jnp.zeros((16,), jnp.float32)
        return carry

    lax.fori_loop(0, 16, fill_zero, 0)

    def fill_zero2(i, carry):
        for j in range(H // 16):
            rows_v[0, i, pl.ds(j * 16, 16)] = jnp.zeros((16,), jnp.float32)
        return carry

    lax.fori_loop(0, _AZR // 4, fill_zero2, 0)
    zd = [
        pltpu.async_copy(
            rows_v.at[0, pl.ds(0, _AZR // 4)],
            acc.at[pl.ds(sid * _AZR + q * (_AZR // 4), _AZR // 4)], sem_s)
        for q in range(4)
    ]
    for d in zd:
        d.wait()
    stage.wait()
    plsc.subcore_barrier()

    def load_idx(c, b):
        base = sid * ROWS_PER_TILE + c * _CR
        pltpu.sync_copy(src_hbm.at[pl.ds(base, _CR)], src_v.at[b])
        pltpu.sync_copy(dst_hbm.at[pl.ds(base, _CR)], dst_v.at[b])
        for j in range(_CR):
            for k in range(128 // 16):
                v = dst_v[b, j, pl.ds(k * 16, 16)] - base_node
                keep = (v >= 0) & (v < HALF)
                dump = HALF + (v & (DUMP - 1))
                dst_v[b, j, pl.ds(k * 16, 16)] = jnp.where(keep, v, dump)

    def fire_gathers(b):
        return [
            pltpu.async_copy(
                g_sh.at[src_v.at[b, j]],
                rows_v.at[b, pl.ds(j * 128, 128)], sem_g)
            for j in range(_CR)
        ]

    def fire_scatters(b):
        return [
            pltpu.async_copy(
                rows_v.at[b, pl.ds(j * 128, 128)],
                acc.at[dst_v.at[b, j]], sem_s, add=True)
            for j in range(_CR)
        ]

    def pipe(c, carry):
        load_idx(2 * c, 0)
        gd0 = fire_gathers(0)
        load_idx(2 * c + 1, 1)
        for d in gd0:
            d.wait()
        sd0 = fire_scatters(0)
        gd1 = fire_gathers(1)
        for d in gd1:
            d.wait()
        for d in sd0:
            d.wait()
        sd1 = fire_scatters(1)
        for d in sd1:
            d.wait()
        return carry

    lax.fori_loop(0, ROWS_PER_TILE // (2 * _CR), pipe, 0)
    plsc.subcore_barrier()
    pltpu.sync_copy(acc.at[pl.ds(sid * (HALF // NS), HALF // NS)],
                    out_hbm.at[pl.ds(base_node + sid * (HALF // NS),
                                     HALF // NS)])


_BLK = 512


def _tc_prep(x_pad, degp, W1):
    def body(deg_ref, x_ref, w_ref, dinv_ref, g_ref):
        deg = deg_ref[0, :, 0:1] + deg_ref[1, :, 0:1] + 1.0
        dinv = lax.rsqrt(deg)
        h = jnp.dot(x_ref[...], w_ref[...], preferred_element_type=jnp.float32)
        dinv_ref[...] = dinv
        g_ref[...] = dinv * h

    return pl.pallas_call(
        body,
        grid=(NPAD // _BLK,),
        in_specs=[
            pl.BlockSpec((NC, _BLK, 16), lambda i: (0, i, 0)),
            pl.BlockSpec((_BLK, D), lambda i: (i, 0)),
            pl.BlockSpec((D, H), lambda i: (0, 0)),
        ],
        out_specs=[
            pl.BlockSpec((_BLK, 1), lambda i: (i, 0)),
            pl.BlockSpec((_BLK, H), lambda i: (i, 0)),
        ],
        out_shape=[
            jax.ShapeDtypeStruct((NPAD, 1), jnp.float32),
            jax.ShapeDtypeStruct((NPAD, H), jnp.float32),
        ],
    )(degp, x_pad, W1)


def _tc_mid(p, g, dinv, b, Wn):
    def body(p_ref, g_ref, dinv_ref, b_ref, w_ref, out_ref):
        dinv = dinv_ref[...]
        h = jnp.maximum(
            dinv * (p_ref[...] + g_ref[...]) + b_ref[...], 0.0)
        out_ref[...] = dinv * jnp.dot(
            h, w_ref[...], preferred_element_type=jnp.float32)

    return pl.pallas_call(
        body,
        grid=(NPAD // _BLK,),
        in_specs=[
            pl.BlockSpec((_BLK, H), lambda i: (i, 0)),
            pl.BlockSpec((_BLK, H), lambda i: (i, 0)),
            pl.BlockSpec((_BLK, 1), lambda i: (i, 0)),
            pl.BlockSpec((1, H), lambda i: (0, 0)),
            pl.BlockSpec((H, H), lambda i: (0, 0)),
        ],
        out_specs=pl.BlockSpec((_BLK, H), lambda i: (i, 0)),
        out_shape=jax.ShapeDtypeStruct((NPAD, H), jnp.float32),
    )(p, g, dinv, b, Wn)


def _tc_final(p, g, dinv, b, Wout, bout):
    def body(p_ref, g_ref, dinv_ref, b_ref, w_ref, bo_ref, out_ref):
        dinv = dinv_ref[...]
        h = jnp.maximum(
            dinv * (p_ref[...] + g_ref[...]) + b_ref[...], 0.0)
        o = jnp.dot(h, w_ref[...], preferred_element_type=jnp.float32)
        out_ref[...] = jnp.maximum(o + bo_ref[...], 0.0)

    return pl.pallas_call(
        body,
        grid=(NPAD // _BLK,),
        in_specs=[
            pl.BlockSpec((_BLK, H), lambda i: (i, 0)),
            pl.BlockSpec((_BLK, H), lambda i: (i, 0)),
            pl.BlockSpec((_BLK, 1), lambda i: (i, 0)),
            pl.BlockSpec((1, H), lambda i: (0, 0)),
            pl.BlockSpec((H, OUT), lambda i: (0, 0)),
            pl.BlockSpec((1, OUT), lambda i: (0, 0)),
        ],
        out_specs=pl.BlockSpec((_BLK, OUT), lambda i: (i, 0)),
        out_shape=jax.ShapeDtypeStruct((NPAD, OUT), jnp.float32),
    )(p, g, dinv, b, Wout, bout)


def kernel(x, edge_index, edge_attr, W1, b1, W2, b2, W3, b3, Wout, bout):
    src = edge_index[0]
    dst = edge_index[1]
    arange_pad = jnp.arange(EPAD - E, dtype=jnp.int32)
    pad_src = arange_pad % N
    pad_dst = N + arange_pad % (NPAD - N)
    src_p = jnp.concatenate([src, pad_src]).reshape(EROWS, 128)
    dst_p = jnp.concatenate([dst, pad_dst]).reshape(EROWS, 128)
    x_pad = jnp.zeros((NPAD, D), jnp.float32).at[:N].set(x)

    degp = _deg_kernel(dst_p)
    dinv, g = _tc_prep(x_pad, degp, W1)

    b1r = b1.reshape(1, H)
    b2r = b2.reshape(1, H)
    b3r = b3.reshape(1, H)
    boutr = bout.reshape(1, OUT)

    p = _msg_kernel(g, src_p, dst_p)
    g = _tc_mid(p, g, dinv, b1r, W2)
    p = _msg_kernel(g, src_p, dst_p)
    g = _tc_mid(p, g, dinv, b2r, W3)
    p = _msg_kernel(g, src_p, dst_p)
    out = _tc_final(p, g, dinv, b3r, Wout, boutr)
    return out[:N]


# speedup vs baseline: 1.2750x; 1.2750x over previous
import functools

import jax
import jax.numpy as jnp
from jax import lax
from jax.experimental import pallas as pl
from jax.experimental.pallas import tpu as pltpu
from jax.experimental.pallas import tpu_sc as plsc

N = 10000
D = 128
H = 64
OUT = 64
E = 320000

NC = 2
NS = 16
NW = NC * NS

NPAD = 10240
HALF = NPAD // NC
DUMP = 128
ACCR = HALF + DUMP
EPAD = 327680
EROWS = EPAD // 128
ROWS_PER_TILE = EROWS // NS
ZROWS = NPAD // NS
GSROWS = N // NS

_mesh = plsc.VectorSubcoreMesh(core_axis_name="c", subcore_axis_name="s")


ESLOT = EPAD // NW
SROWS = ESLOT // 128


@functools.partial(
    pl.kernel,
    out_type=[
        jax.ShapeDtypeStruct((NC, NW, SROWS, 128), jnp.int32),
        jax.ShapeDtypeStruct((NC, NW, SROWS, 128), jnp.int32),
        jax.ShapeDtypeStruct((NC * NW * 16,), jnp.int32),
        jax.ShapeDtypeStruct((NC, NPAD, 16), jnp.float32),
    ],
    mesh=_mesh,
    scratch_types=[
        pltpu.VMEM((8, 128), jnp.int32),
        pltpu.VMEM((8, 128), jnp.int32),
        pltpu.VMEM((SROWS + 1, 128), jnp.int32),
        pltpu.VMEM((SROWS + 1, 128), jnp.int32),
        pltpu.VMEM((SROWS + 1, 128), jnp.int32),
        pltpu.VMEM((SROWS + 1, 128), jnp.int32),
        pltpu.VMEM((16,), jnp.int32),
        pltpu.VMEM((128, 16), jnp.float32),
        pltpu.VMEM((64, 16), jnp.float32),
        pltpu.VMEM_SHARED((NPAD, 16), jnp.float32),
    ],
    compiler_params=pltpu.CompilerParams(use_tc_tiling_on_sc=False, needs_layout_passes=False),
)
def _bucket_kernel(src_hbm, dst_hbm, bsrc_hbm, bdst_hbm, cnt_hbm, deg_hbm,
                   src_v, dst_v, b0s, b0d, b1s, b1d, cnt_v, ones_v, zero_v,
                   acc):
    cid = lax.axis_index("c")
    sid = lax.axis_index("s")
    wid = cid * NS + sid
    iota = lax.broadcasted_iota(jnp.int32, (16,), 0)

    def fill_ones(i, carry):
        ones_v[i, :] = jnp.ones((16,), jnp.float32)
        return carry

    lax.fori_loop(0, 128, fill_ones, 0)

    def fill_zero(i, carry):
        zero_v[i, :] = jnp.zeros((16,), jnp.float32)
        return carry

    lax.fori_loop(0, 64, fill_zero, 0)

    def zero_acc(i, carry):
        pltpu.sync_copy(zero_v, acc.at[pl.ds(sid * ZROWS + i * 64, 64)])
        return carry

    lax.fori_loop(0, ZROWS // 64, zero_acc, 0)
    plsc.subcore_barrier()

    def scat(bs, bd, pos, s16, d16, m):
        pos = jnp.maximum(pos, 0)
        r = lax.shift_right_logical(pos, 7)
        col = pos & 127
        plsc.store_scatter(bs, [r, col], s16, mask=m)
        plsc.store_scatter(bd, [r, col], d16, mask=m)

    def chunk(c, offs):
        base = wid * SROWS + c * 8
        pltpu.sync_copy(src_hbm.at[pl.ds(base, 8)], src_v)
        pltpu.sync_copy(dst_hbm.at[pl.ds(base, 8)], dst_v)
        for j in range(8):
            pltpu.sync_copy(ones_v, acc.at[dst_v.at[j]], add=True)

        def row(j, offs2):
            off0, off1 = offs2
            for k in range(8):
                s16 = src_v[j, pl.ds(k * 16, 16)]
                d16 = dst_v[j, pl.ds(k * 16, 16)]
                m0 = d16 < HALF
                n0 = jnp.sum(m0.astype(jnp.int32))
                k0 = jnp.where(m0, iota, iota + 16)
                _, ss0 = plsc.sort_key_val(k0, s16)
                _, dd0 = plsc.sort_key_val(k0, d16)
                scat(b0s, b0d, off0 + iota, ss0, dd0, None)
                k1 = jnp.where(m0, iota + 16, iota)
                _, ss1 = plsc.sort_key_val(k1, s16)
                _, dd1 = plsc.sort_key_val(k1, d16)
                scat(b1s, b1d, off1 + iota, ss1, dd1, None)
                off0 = off0 + n0
                off1 = off1 + (16 - n0)
            return (off0, off1)

        return lax.fori_loop(0, 8, row, offs)

    off0, off1 = lax.fori_loop(0, SROWS // 8, chunk,
                               (jnp.int32(0), jnp.int32(0)))

    def pad_bucket(bs, bd, off, core):
        end = (off + 1023) & ~jnp.int32(1023)

        def fill(t, carry):
            base = off + t * 16
            pos = base + iota
            vs = pos & 4095
            vd = (1 - core) * HALF + (pos & (DUMP - 1))
            scat(bs, bd, pos, vs, vd, None)
            return carry

        lax.fori_loop(0, (end - off + 15) // 16, fill, 0)
        return lax.shift_right_logical(end, 7)

    rows0 = pad_bucket(b0s, b0d, off0, 0)
    rows1 = pad_bucket(b1s, b1d, off1, 1)

    pltpu.sync_copy(b0s.at[pl.ds(0, SROWS)], bsrc_hbm.at[0, wid])
    pltpu.sync_copy(b0d.at[pl.ds(0, SROWS)], bdst_hbm.at[0, wid])
    pltpu.sync_copy(b1s.at[pl.ds(0, SROWS)], bsrc_hbm.at[1, wid])
    pltpu.sync_copy(b1d.at[pl.ds(0, SROWS)], bdst_hbm.at[1, wid])
    cnt_v[...] = jnp.full((16,), rows0, jnp.int32)
    pltpu.sync_copy(cnt_v, cnt_hbm.at[pl.ds((0 * NW + wid) * 16, 16)])
    cnt_v[...] = jnp.full((16,), rows1, jnp.int32)
    pltpu.sync_copy(cnt_v, cnt_hbm.at[pl.ds((1 * NW + wid) * 16, 16)])
    plsc.subcore_barrier()
    pltpu.sync_copy(acc.at[pl.ds(sid * ZROWS, ZROWS)],
                    deg_hbm.at[cid, pl.ds(sid * ZROWS, ZROWS)])


_CR = 4
_NBUF = 2
_AZR = ACCR // NS


@functools.partial(
    pl.kernel,
    out_type=jax.ShapeDtypeStruct((NPAD, H), jnp.float32),
    mesh=_mesh,
    scratch_types=[
        pltpu.VMEM((_NBUF, _CR, 128), jnp.int32),
        pltpu.VMEM((_NBUF, _CR, 128), jnp.int32),
        pltpu.VMEM((_NBUF, _CR * 128, H), jnp.float32),
        pltpu.VMEM((16, H), jnp.float32),
        pltpu.VMEM_SHARED((ACCR, H), jnp.float32),
        pltpu.VMEM_SHARED((N, H), jnp.float32),
        pltpu.SemaphoreType.DMA,
        pltpu.SemaphoreType.DMA,
        pltpu.SemaphoreType.DMA,
        pltpu.VMEM((NC * NW * 16,), jnp.int32),
    ],
    compiler_params=pltpu.CompilerParams(use_tc_tiling_on_sc=False, needs_layout_passes=False),
)
def _msg_kernel(g_hbm, src_hbm, dst_hbm, cnt_hbm, out_hbm,
                src_v, dst_v, rows_v, zero_v, acc, g_sh, sem_g, sem_s, sem_t,
                cnt_v):
    cid = lax.axis_index("c")
    sid = lax.axis_index("s")
    base_node = cid * HALF
    pltpu.sync_copy(cnt_hbm, cnt_v)

    stage = pltpu.async_copy(
        g_hbm.at[pl.ds(sid * GSROWS, GSROWS)],
        g_sh.at[pl.ds(sid * GSROWS, GSROWS)], sem_t)

    def fill_zero(i, carry):
        for j in range(H // 16):
            zero_v[i, pl.ds(j * 16, 16)] = jnp.zeros((16,), jnp.float32)
        return carry

    lax.fori_loop(0, 16, fill_zero, 0)

    def fill_zero2(i, carry):
        for j in range(H // 16):
            rows_v[0, i, pl.ds(j * 16, 16)] = jnp.zeros((16,), jnp.float32)
        return carry

    lax.fori_loop(0, _AZR // 4, fill_zero2, 0)
    zd = [
        pltpu.async_copy(
            rows_v.at[0, pl.ds(0, _AZR // 4)],
            acc.at[pl.ds(sid * _AZR + q * (_AZR // 4), _AZR // 4)], sem_s)
        for q in range(4)
    ]
    for d in zd:
        d.wait()
    stage.wait()
    plsc.subcore_barrier()

    def load_idx(slot, c, b):
        base = c * _CR
        pltpu.sync_copy(src_hbm.at[cid, slot, pl.ds(base, _CR)], src_v.at[b])
        pltpu.sync_copy(dst_hbm.at[cid, slot, pl.ds(base, _CR)], dst_v.at[b])
        for j in range(_CR):
            for k in range(128 // 16):
                v = dst_v[b, j, pl.ds(k * 16, 16)] - base_node
                keep = (v >= 0) & (v < HALF)
                dump = HALF + (v & (DUMP - 1))
                dst_v[b, j, pl.ds(k * 16, 16)] = jnp.where(keep, v, dump)

    def fire_gathers(b):
        return [
            pltpu.async_copy(
                g_sh.at[src_v.at[b, j]],
                rows_v.at[b, pl.ds(j * 128, 128)], sem_g)
            for j in range(_CR)
        ]

    def fire_scatters(b):
        return [
            pltpu.async_copy(
                rows_v.at[b, pl.ds(j * 128, 128)],
                acc.at[dst_v.at[b, j]], sem_s, add=True)
            for j in range(_CR)
        ]

    def pipe_slot(slot, c):
        load_idx(slot, 2 * c, 0)
        gd0 = fire_gathers(0)
        load_idx(slot, 2 * c + 1, 1)
        for d in gd0:
            d.wait()
        sd0 = fire_scatters(0)
        gd1 = fire_gathers(1)
        for d in gd1:
            d.wait()
        for d in sd0:
            d.wait()
        sd1 = fire_scatters(1)
        for d in sd1:
            d.wait()

    for t in range(2):
        slot = 2 * sid + t
        rows_n = cnt_v[pl.ds((cid * NW + slot) * 16, 16)][0]

        def pipe(c, carry, _slot=slot):
            pipe_slot(_slot, c)
            return carry

        lax.fori_loop(0, rows_n // (2 * _CR), pipe, 0)
    plsc.subcore_barrier()
    pltpu.sync_copy(acc.at[pl.ds(sid * (HALF // NS), HALF // NS)],
                    out_hbm.at[pl.ds(base_node + sid * (HALF // NS),
                                     HALF // NS)])


_BLK = 512


def _tc_prep(x_pad, degp, W1):
    def body(deg_ref, x_ref, w_ref, dinv_ref, g_ref):
        deg = deg_ref[0, :, 0:1] + deg_ref[1, :, 0:1] + 1.0
        dinv = lax.rsqrt(deg)
        h = jnp.dot(x_ref[...], w_ref[...], preferred_element_type=jnp.float32)
        dinv_ref[...] = dinv
        g_ref[...] = dinv * h

    return pl.pallas_call(
        body,
        grid=(NPAD // _BLK,),
        in_specs=[
            pl.BlockSpec((NC, _BLK, 16), lambda i: (0, i, 0)),
            pl.BlockSpec((_BLK, D), lambda i: (i, 0)),
            pl.BlockSpec((D, H), lambda i: (0, 0)),
        ],
        out_specs=[
            pl.BlockSpec((_BLK, 1), lambda i: (i, 0)),
            pl.BlockSpec((_BLK, H), lambda i: (i, 0)),
        ],
        out_shape=[
            jax.ShapeDtypeStruct((NPAD, 1), jnp.float32),
            jax.ShapeDtypeStruct((NPAD, H), jnp.float32),
        ],
    )(degp, x_pad, W1)


def _tc_mid(p, g, dinv, b, Wn):
    def body(p_ref, g_ref, dinv_ref, b_ref, w_ref, out_ref):
        dinv = dinv_ref[...]
        h = jnp.maximum(
            dinv * (p_ref[...] + g_ref[...]) + b_ref[...], 0.0)
        out_ref[...] = dinv * jnp.dot(
            h, w_ref[...], preferred_element_type=jnp.float32)

    return pl.pallas_call(
        body,
        grid=(NPAD // _BLK,),
        in_specs=[
            pl.BlockSpec((_BLK, H), lambda i: (i, 0)),
            pl.BlockSpec((_BLK, H), lambda i: (i, 0)),
            pl.BlockSpec((_BLK, 1), lambda i: (i, 0)),
            pl.BlockSpec((1, H), lambda i: (0, 0)),
            pl.BlockSpec((H, H), lambda i: (0, 0)),
        ],
        out_specs=pl.BlockSpec((_BLK, H), lambda i: (i, 0)),
        out_shape=jax.ShapeDtypeStruct((NPAD, H), jnp.float32),
    )(p, g, dinv, b, Wn)


def _tc_final(p, g, dinv, b, Wout, bout):
    def body(p_ref, g_ref, dinv_ref, b_ref, w_ref, bo_ref, out_ref):
        dinv = dinv_ref[...]
        h = jnp.maximum(
            dinv * (p_ref[...] + g_ref[...]) + b_ref[...], 0.0)
        o = jnp.dot(h, w_ref[...], preferred_element_type=jnp.float32)
        out_ref[...] = jnp.maximum(o + bo_ref[...], 0.0)

    return pl.pallas_call(
        body,
        grid=(NPAD // _BLK,),
        in_specs=[
            pl.BlockSpec((_BLK, H), lambda i: (i, 0)),
            pl.BlockSpec((_BLK, H), lambda i: (i, 0)),
            pl.BlockSpec((_BLK, 1), lambda i: (i, 0)),
            pl.BlockSpec((1, H), lambda i: (0, 0)),
            pl.BlockSpec((H, OUT), lambda i: (0, 0)),
            pl.BlockSpec((1, OUT), lambda i: (0, 0)),
        ],
        out_specs=pl.BlockSpec((_BLK, OUT), lambda i: (i, 0)),
        out_shape=jax.ShapeDtypeStruct((NPAD, OUT), jnp.float32),
    )(p, g, dinv, b, Wout, bout)


def kernel(x, edge_index, edge_attr, W1, b1, W2, b2, W3, b3, Wout, bout):
    src = edge_index[0]
    dst = edge_index[1]
    arange_pad = jnp.arange(EPAD - E, dtype=jnp.int32)
    pad_src = arange_pad % N
    pad_dst = N + arange_pad % (NPAD - N)
    src_p = jnp.concatenate([src, pad_src]).reshape(EROWS, 128)
    dst_p = jnp.concatenate([dst, pad_dst]).reshape(EROWS, 128)
    x_pad = jnp.zeros((NPAD, D), jnp.float32).at[:N].set(x)

    bsrc, bdst, cnts, degp = _bucket_kernel(src_p, dst_p)
    dinv, g = _tc_prep(x_pad, degp, W1)

    b1r = b1.reshape(1, H)
    b2r = b2.reshape(1, H)
    b3r = b3.reshape(1, H)
    boutr = bout.reshape(1, OUT)

    p = _msg_kernel(g, bsrc, bdst, cnts)
    g = _tc_mid(p, g, dinv, b1r, W2)
    p = _msg_kernel(g, bsrc, bdst, cnts)
    g = _tc_mid(p, g, dinv, b2r, W3)
    p = _msg_kernel(g, bsrc, bdst, cnts)
    out = _tc_final(p, g, dinv, b3r, Wout, boutr)
    return out[:N]
